# Initial kernel scaffold; baseline (speedup 1.0000x reference)
import jax
import jax.numpy as jnp
from jax.experimental import pallas as pl


def kernel(x, f0, f1, f2):
    raise NotImplementedError("write your pallas kernel here")


# speedup vs baseline: 21.4446x; 21.4446x over previous
import functools
import math

import jax
import jax.numpy as jnp
from jax import lax
from jax.experimental import pallas as pl
from jax.experimental.pallas import tpu as pltpu, tpu_sc as plsc

N_PTS = 262144
N_FEAT = 8
RES = (32, 64, 128)
L = 4
N_OUT = 2 * L * 3 + 3 * N_FEAT

NW = 32
PPW = N_PTS // NW
CP = 128
NCHUNK = PPW // CP
NGROUP = CP // 16

PIO2 = math.pi * 0.5
INV_PIO2 = 2.0 / math.pi
S1, S2, S3 = -0.16666667, 0.0083333310, -0.00019840874
C1, C2, C3 = -0.5, 0.041666638, -0.0013888378


def _splat(v):
    return jnp.full((16,), v, jnp.int32)


def _body(xflat, t0, t1, t2, out, xbuf, idxbuf, wbuf, d0, d1, d2, staging, sem):
    wid = lax.axis_index("s") * 2 + lax.axis_index("c")
    base = wid * PPW
    for j in range(3):
        pltpu.sync_copy(xflat.at[pl.ds(j * N_PTS + base, PPW)],
                        xbuf.at[pl.ds(j * PPW, PPW)])

    iota = lax.iota(jnp.int32, 16)
    tables = (t0, t1, t2)
    dests = (d0, d1, d2)

    @pl.loop(0, NCHUNK)
    def chunk(ci):
        xoff = ci * CP

        def idx_group(g, carry):
            off = xoff + g * 16
            xv = [xbuf[pl.ds(j * PPW + off, 16)] for j in range(3)]
            xp = [v + 1.0 for v in xv]
            for gi in range(3):
                r = RES[gi]
                s = 0.5 * (r - 1)
                p = [v * s for v in xp]
                f = [v.astype(jnp.int32) for v in p]
                w = [p[j] - f[j].astype(jnp.float32) for j in range(3)]
                i0 = [jnp.minimum(v, r - 1) for v in f]
                i1 = [jnp.minimum(v + 1, r - 1) for v in i0]
                for j in range(3):
                    wbuf[pl.ds((3 * gi + j) * CP + g * 16, 16)] = w[j]
                zb = (i0[2] * (r * r), i1[2] * (r * r))
                yb = (i0[1] * r, i1[1] * r)
                xb = (i0[0], i1[0])
                k = 0
                for dz in range(2):
                    for dy in range(2):
                        for dx in range(2):
                            idxbuf[pl.ds((gi * 8 + k) * CP + g * 16, 16)] = (
                                zb[dz] + yb[dy] + xb[dx])
                            k += 1
            return carry

        lax.fori_loop(0, NGROUP, idx_group, 0)

        descs = []
        for gi in range(3):
            for k in range(8):
                descs.append(pltpu.async_copy(
                    tables[gi].at[idxbuf.at[pl.ds((gi * 8 + k) * CP, CP)]],
                    dests[gi].at[k], sem))
        for d in descs:
            d.wait()

        def blend_group(g, carry):
            off = xoff + g * 16
            rowv = g * 16 + iota
            rowv48 = rowv * 48
            xv = [xbuf[pl.ds(j * PPW + off, 16)] for j in range(3)]
            for l in range(L):
                coef = (l + 1) * PIO2
                for j in range(3):
                    arg = xv[j] * coef
                    t = arg * INV_PIO2
                    k8 = (t + 8.5).astype(jnp.int32)
                    kf = k8.astype(jnp.float32) - 8.0
                    rr = arg - kf * PIO2
                    r2 = rr * rr
                    sp = rr + rr * (r2 * (S1 + r2 * (S2 + r2 * S3)))
                    cp = 1.0 + r2 * (C1 + r2 * (C2 + r2 * C3))
                    swap = (k8 & 1) == 1
                    sv = jnp.where(swap, cp, sp)
                    cv = jnp.where(swap, sp, cp)
                    so = jnp.where((k8 & 2) == 2, -sv, sv)
                    co = jnp.where(((k8 + 1) & 2) == 2, -cv, cv)
                    col = l * 3 + j
                    plsc.store_scatter(staging, [rowv48 + col], so)
                    plsc.store_scatter(staging, [rowv48 + (12 + col)], co)
            for gi in range(3):
                wx = wbuf[pl.ds((3 * gi + 0) * CP + g * 16, 16)]
                wy = wbuf[pl.ds((3 * gi + 1) * CP + g * 16, 16)]
                wz = wbuf[pl.ds((3 * gi + 2) * CP + g * 16, 16)]
                dg = dests[gi]
                for ch in range(8):
                    chv = _splat(ch)
                    c = [plsc.load_gather(dg, [_splat(k), rowv, chv])
                         for k in range(8)]
                    c00 = c[0] + wx * (c[1] - c[0])
                    c01 = c[2] + wx * (c[3] - c[2])
                    c10 = c[4] + wx * (c[5] - c[4])
                    c11 = c[6] + wx * (c[7] - c[6])
                    c0 = c00 + wy * (c01 - c00)
                    c1 = c10 + wy * (c11 - c10)
                    o = c0 + wz * (c1 - c0)
                    plsc.store_scatter(staging, [rowv48 + (24 + 8 * gi + ch)], o)
            return carry

        lax.fori_loop(0, NGROUP, blend_group, 0)

        pltpu.sync_copy(staging,
                        out.at[pl.ds((base + xoff) * N_OUT, CP * N_OUT)])


@functools.cache
def _build():
    mesh = plsc.VectorSubcoreMesh(core_axis_name="c", subcore_axis_name="s")
    return pl.kernel(
        _body,
        out_type=jax.ShapeDtypeStruct((N_PTS * N_OUT,), jnp.float32),
        mesh=mesh,
        compiler_params=pltpu.CompilerParams(
            needs_layout_passes=False, use_tc_tiling_on_sc=False),
        scratch_types=[
            pltpu.VMEM((3 * PPW,), jnp.float32),
            pltpu.VMEM((24 * CP,), jnp.int32),
            pltpu.VMEM((9 * CP,), jnp.float32),
            pltpu.VMEM((8, CP, N_FEAT), jnp.float32),
            pltpu.VMEM((8, CP, N_FEAT), jnp.float32),
            pltpu.VMEM((8, CP, N_FEAT), jnp.float32),
            pltpu.VMEM((CP * N_OUT,), jnp.float32),
            pltpu.SemaphoreType.DMA,
        ],
    )


def kernel(x, f0, f1, f2):
    xflat = x.T.reshape(-1)
    tabs = [jnp.transpose(f[0], (1, 2, 3, 0)).reshape(-1, N_FEAT)
            for f in (f0, f1, f2)]
    out = _build()(xflat, *tabs)
    return out.reshape(N_PTS, N_OUT)


# speedup vs baseline: 28.0685x; 1.3089x over previous
import functools
import math

import jax
import jax.numpy as jnp
from jax import lax
from jax.experimental import pallas as pl
from jax.experimental.pallas import tpu as pltpu, tpu_sc as plsc

N_PTS = 262144
N_FEAT = 8
RES = (32, 64, 128)
NVOX = tuple(r * r * r for r in RES)
OFS = (0, NVOX[0], NVOX[0] + NVOX[1])
NROWS = NVOX[0] + NVOX[1] + NVOX[2]
L = 4
N_OUT = 2 * L * 3 + 3 * N_FEAT

NW = 32
PPW = N_PTS // NW
CP = 128
NCHUNK = PPW // CP
NGROUP = CP // 16

PIO2 = math.pi * 0.5
INV_PIO2 = 2.0 / math.pi
S1, S2, S3 = -0.16666667, 0.0083333310, -0.00019840874
C1, C2, C3 = -0.5, 0.041666638, -0.0013888378

_SC_PARAMS = dict(
    compiler_params=pltpu.CompilerParams(
        needs_layout_passes=False, use_tc_tiling_on_sc=False),
)


def _wid():
    return lax.axis_index("s") * 2 + lax.axis_index("c")


def _relayout_body(ff0, ff1, ff2, tout, chbuf, obuf):
    wid = _wid()
    iota = lax.iota(jnp.int32, 16)
    lane_c = iota & 7
    lane_p = lax.shift_right_logical(iota, 3)
    srcs = (ff0, ff1, ff2)
    for gi in range(3):
        nv = NVOX[gi]
        span = nv // NW
        k = min(span, 4096)
        nck = span // k
        const = lane_c * k + lane_p
        wbase = wid * span

        @pl.loop(0, nck)
        def _chunk(ci, _gi=gi, _k=k, _const=const, _wbase=wbase, _nv=nv):
            pos0 = _wbase + ci * _k
            for ch in range(8):
                pltpu.sync_copy(
                    srcs[_gi].at[pl.ds(ch * _nv + pos0, _k)],
                    chbuf.at[pl.ds(ch * _k, _k)])

            def _pair(j, carry):
                v = plsc.load_gather(chbuf, [_const + j * 2])
                obuf[pl.ds(j * 16, 16)] = v
                return carry

            lax.fori_loop(0, _k // 2, _pair, 0)
            pltpu.sync_copy(
                obuf.at[pl.ds(0, _k * 8)],
                tout.at[pl.ds((OFS[_gi] + pos0) * 8, _k * 8)])


def _splat(v):
    return jnp.full((16,), v, jnp.int32)


def _main_body(xflat, tab, out, xbuf, idxbuf, wbuf, d0, d1, d2, staging, sem):
    base = _wid() * PPW
    for j in range(3):
        pltpu.sync_copy(xflat.at[pl.ds(j * N_PTS + base, PPW)],
                        xbuf.at[pl.ds(j * PPW, PPW)])

    iota = lax.iota(jnp.int32, 16)
    dests = (d0, d1, d2)

    @pl.loop(0, NCHUNK)
    def chunk(ci):
        xoff = ci * CP

        def idx_group(g, carry):
            off = xoff + g * 16
            xv = [xbuf[pl.ds(j * PPW + off, 16)] for j in range(3)]
            xp = [v + 1.0 for v in xv]
            for gi in range(3):
                r = RES[gi]
                s = 0.5 * (r - 1)
                p = [v * s for v in xp]
                f = [v.astype(jnp.int32) for v in p]
                w = [p[j] - f[j].astype(jnp.float32) for j in range(3)]
                i0 = [jnp.minimum(v, r - 1) for v in f]
                i1 = [jnp.minimum(v + 1, r - 1) for v in i0]
                for j in range(3):
                    wbuf[pl.ds((3 * gi + j) * CP + g * 16, 16)] = w[j]
                zb = (i0[2] * (r * r) + OFS[gi], i1[2] * (r * r) + OFS[gi])
                yb = (i0[1] * r, i1[1] * r)
                xb = (i0[0], i1[0])
                k = 0
                for dz in range(2):
                    for dy in range(2):
                        for dx in range(2):
                            idxbuf[pl.ds((gi * 8 + k) * CP + g * 16, 16)] = (
                                zb[dz] + yb[dy] + xb[dx])
                            k += 1
            return carry

        lax.fori_loop(0, NGROUP, idx_group, 0)

        descs = []
        for gi in range(3):
            for k in range(8):
                descs.append(pltpu.async_copy(
                    tab.at[idxbuf.at[pl.ds((gi * 8 + k) * CP, CP)]],
                    dests[gi].at[k], sem))
        for d in descs:
            d.wait()

        def blend_group(g, carry):
            off = xoff + g * 16
            rowv = g * 16 + iota
            rowv48 = rowv * 48
            xv = [xbuf[pl.ds(j * PPW + off, 16)] for j in range(3)]
            for l in range(L):
                coef = (l + 1) * PIO2
                for j in range(3):
                    arg = xv[j] * coef
                    t = arg * INV_PIO2
                    k8 = (t + 8.5).astype(jnp.int32)
                    kf = k8.astype(jnp.float32) - 8.0
                    rr = arg - kf * PIO2
                    r2 = rr * rr
                    sp = rr + rr * (r2 * (S1 + r2 * (S2 + r2 * S3)))
                    cp = 1.0 + r2 * (C1 + r2 * (C2 + r2 * C3))
                    swap = (k8 & 1) == 1
                    sv = jnp.where(swap, cp, sp)
                    cv = jnp.where(swap, sp, cp)
                    so = jnp.where((k8 & 2) == 2, -sv, sv)
                    co = jnp.where(((k8 + 1) & 2) == 2, -cv, cv)
                    col = l * 3 + j
                    plsc.store_scatter(staging, [rowv48 + col], so)
                    plsc.store_scatter(staging, [rowv48 + (12 + col)], co)
            for gi in range(3):
                wx = wbuf[pl.ds((3 * gi + 0) * CP + g * 16, 16)]
                wy = wbuf[pl.ds((3 * gi + 1) * CP + g * 16, 16)]
                wz = wbuf[pl.ds((3 * gi + 2) * CP + g * 16, 16)]
                dg = dests[gi]
                for ch in range(8):
                    chv = _splat(ch)
                    c = [plsc.load_gather(dg, [_splat(k), rowv, chv])
                         for k in range(8)]
                    c00 = c[0] + wx * (c[1] - c[0])
                    c01 = c[2] + wx * (c[3] - c[2])
                    c10 = c[4] + wx * (c[5] - c[4])
                    c11 = c[6] + wx * (c[7] - c[6])
                    c0 = c00 + wy * (c01 - c00)
                    c1 = c10 + wy * (c11 - c10)
                    o = c0 + wz * (c1 - c0)
                    plsc.store_scatter(staging, [rowv48 + (24 + 8 * gi + ch)], o)
            return carry

        lax.fori_loop(0, NGROUP, blend_group, 0)

        pltpu.sync_copy(staging,
                        out.at[pl.ds((base + xoff) * N_OUT, CP * N_OUT)])


@functools.cache
def _build():
    mesh = plsc.VectorSubcoreMesh(core_axis_name="c", subcore_axis_name="s")
    relayout = pl.kernel(
        _relayout_body,
        out_type=jax.ShapeDtypeStruct((NROWS * N_FEAT,), jnp.float32),
        mesh=mesh,
        scratch_types=[
            pltpu.VMEM((8 * 4096,), jnp.float32),
            pltpu.VMEM((4096 * 8,), jnp.float32),
        ],
        **_SC_PARAMS,
    )
    main = pl.kernel(
        _main_body,
        out_type=jax.ShapeDtypeStruct((N_PTS * N_OUT,), jnp.float32),
        mesh=mesh,
        scratch_types=[
            pltpu.VMEM((3 * PPW,), jnp.float32),
            pltpu.VMEM((24 * CP,), jnp.int32),
            pltpu.VMEM((9 * CP,), jnp.float32),
            pltpu.VMEM((8, CP, N_FEAT), jnp.float32),
            pltpu.VMEM((8, CP, N_FEAT), jnp.float32),
            pltpu.VMEM((8, CP, N_FEAT), jnp.float32),
            pltpu.VMEM((CP * N_OUT,), jnp.float32),
            pltpu.SemaphoreType.DMA,
        ],
        **_SC_PARAMS,
    )
    return relayout, main


def kernel(x, f0, f1, f2):
    relayout, main = _build()
    xflat = x.T.reshape(-1)
    ffs = [f.reshape(-1) for f in (f0, f1, f2)]
    tab = relayout(*ffs).reshape(NROWS, N_FEAT)
    out = main(xflat, tab)
    return out.reshape(N_PTS, N_OUT)


# speedup vs baseline: 30.1948x; 1.0758x over previous
import functools
import math

import jax
import jax.numpy as jnp
from jax import lax
from jax.experimental import pallas as pl
from jax.experimental.pallas import tpu as pltpu, tpu_sc as plsc

N_PTS = 262144
N_FEAT = 8
RES = (32, 64, 128)
NVOX = tuple(r * r * r for r in RES)
OFS = (0, NVOX[0], NVOX[0] + NVOX[1])
NROWS = NVOX[0] + NVOX[1] + NVOX[2]
L = 4
N_OUT = 2 * L * 3 + 3 * N_FEAT

NW = 32
PPW = N_PTS // NW
CP = 256
NCHUNK = PPW // CP
NGROUP = CP // 16

PIO2 = math.pi * 0.5
INV_PIO2 = 2.0 / math.pi
S1, S2, S3 = -0.16666667, 0.0083333310, -0.00019840874
C1, C2, C3 = -0.5, 0.041666638, -0.0013888378

_SC_PARAMS = dict(
    compiler_params=pltpu.CompilerParams(
        needs_layout_passes=False, use_tc_tiling_on_sc=False),
)


def _wid():
    return lax.axis_index("s") * 2 + lax.axis_index("c")


def _relayout_body(ff0, ff1, ff2, tout, chbuf, obuf, rsem):
    wid = _wid()
    iota = lax.iota(jnp.int32, 16)
    lane_c = iota & 7
    lane_p = lax.shift_right_logical(iota, 3)
    srcs = (ff0, ff1, ff2)
    for gi in range(3):
        nv = NVOX[gi]
        span = nv // NW
        k = min(span, 4096)
        nck = span // k
        const = lane_c * k + lane_p
        wbase = wid * span

        @pl.loop(0, nck)
        def _chunk(ci, _gi=gi, _k=k, _const=const, _wbase=wbase, _nv=nv):
            pos0 = _wbase + ci * _k
            descs = [pltpu.async_copy(
                srcs[_gi].at[pl.ds(ch * _nv + pos0, _k)],
                chbuf.at[pl.ds(ch * _k, _k)], rsem) for ch in range(8)]
            for d in descs:
                d.wait()

            @pl.loop(0, _k // 16, unroll=4)
            def _pair(j):
                for u in range(8):
                    v = plsc.load_gather(chbuf, [_const + (j * 16 + u * 2)])
                    obuf[pl.ds(j * 128 + u * 16, 16)] = v

            pltpu.sync_copy(
                obuf.at[pl.ds(0, _k * 8)],
                tout.at[pl.ds((OFS[_gi] + pos0) * 8, _k * 8)])


def _splat(v):
    return jnp.full((16,), v, jnp.int32)


def _main_body(xflat, tab, out, xbuf, idxbuf, wbuf, d0, d1, d2, staging, sem):
    base = _wid() * PPW
    for j in range(3):
        pltpu.sync_copy(xflat.at[pl.ds(j * N_PTS + base, PPW)],
                        xbuf.at[pl.ds(j * PPW, PPW)])

    iota = lax.iota(jnp.int32, 16)
    dests = (d0, d1, d2)

    @pl.loop(0, NCHUNK)
    def chunk(ci):
        xoff = ci * CP

        def idx_group(g, carry):
            off = xoff + g * 16
            xv = [xbuf[pl.ds(j * PPW + off, 16)] for j in range(3)]
            xp = [v + 1.0 for v in xv]
            for gi in range(3):
                r = RES[gi]
                s = 0.5 * (r - 1)
                p = [v * s for v in xp]
                f = [v.astype(jnp.int32) for v in p]
                w = [p[j] - f[j].astype(jnp.float32) for j in range(3)]
                i0 = [jnp.minimum(v, r - 1) for v in f]
                i1 = [jnp.minimum(v + 1, r - 1) for v in i0]
                for j in range(3):
                    wbuf[pl.ds((3 * gi + j) * CP + g * 16, 16)] = w[j]
                zb = (i0[2] * (r * r) + OFS[gi], i1[2] * (r * r) + OFS[gi])
                yb = (i0[1] * r, i1[1] * r)
                xb = (i0[0], i1[0])
                k = 0
                for dz in range(2):
                    for dy in range(2):
                        for dx in range(2):
                            idxbuf[pl.ds((gi * 8 + k) * CP + g * 16, 16)] = (
                                zb[dz] + yb[dy] + xb[dx])
                            k += 1
            return carry

        lax.fori_loop(0, NGROUP, idx_group, 0)

        descs = []
        for gi in range(3):
            for k in range(8):
                descs.append(pltpu.async_copy(
                    tab.at[idxbuf.at[pl.ds((gi * 8 + k) * CP, CP)]],
                    dests[gi].at[k], sem))
        for d in descs:
            d.wait()

        def blend_group(g, carry):
            off = xoff + g * 16
            rowv = g * 16 + iota
            xv = [xbuf[pl.ds(j * PPW + off, 16)] for j in range(3)]
            for l in range(L):
                coef = (l + 1) * PIO2
                for j in range(3):
                    arg = xv[j] * coef
                    t = arg * INV_PIO2
                    k8 = (t + 8.5).astype(jnp.int32)
                    kf = k8.astype(jnp.float32) - 8.0
                    rr = arg - kf * PIO2
                    r2 = rr * rr
                    sp = rr + rr * (r2 * (S1 + r2 * (S2 + r2 * S3)))
                    cp = 1.0 + r2 * (C1 + r2 * (C2 + r2 * C3))
                    swap = (k8 & 1) == 1
                    sv = jnp.where(swap, cp, sp)
                    cv = jnp.where(swap, sp, cp)
                    so = jnp.where((k8 & 2) == 2, -sv, sv)
                    co = jnp.where(((k8 + 1) & 2) == 2, -cv, cv)
                    col = l * 3 + j
                    plsc.store_scatter(staging, [rowv, _splat(col)], so)
                    plsc.store_scatter(staging, [rowv, _splat(12 + col)], co)
            for gi in range(3):
                wx = wbuf[pl.ds((3 * gi + 0) * CP + g * 16, 16)]
                wy = wbuf[pl.ds((3 * gi + 1) * CP + g * 16, 16)]
                wz = wbuf[pl.ds((3 * gi + 2) * CP + g * 16, 16)]
                dg = dests[gi]
                for ch in range(8):
                    chv = _splat(ch)
                    c = [plsc.load_gather(dg, [_splat(k), rowv, chv])
                         for k in range(8)]
                    c00 = c[0] + wx * (c[1] - c[0])
                    c01 = c[2] + wx * (c[3] - c[2])
                    c10 = c[4] + wx * (c[5] - c[4])
                    c11 = c[6] + wx * (c[7] - c[6])
                    c0 = c00 + wy * (c01 - c00)
                    c1 = c10 + wy * (c11 - c10)
                    o = c0 + wz * (c1 - c0)
                    plsc.store_scatter(
                        staging, [rowv, _splat(24 + 8 * gi + ch)], o)
            return carry

        lax.fori_loop(0, NGROUP, blend_group, 0)

        pltpu.sync_copy(staging, out.at[pl.ds(base + xoff, CP)])


@functools.cache
def _build():
    mesh = plsc.VectorSubcoreMesh(core_axis_name="c", subcore_axis_name="s")
    relayout = pl.kernel(
        _relayout_body,
        out_type=jax.ShapeDtypeStruct((NROWS * N_FEAT,), jnp.float32),
        mesh=mesh,
        scratch_types=[
            pltpu.VMEM((8 * 4096,), jnp.float32),
            pltpu.VMEM((4096 * 8,), jnp.float32),
            pltpu.SemaphoreType.DMA,
        ],
        **_SC_PARAMS,
    )
    main = pl.kernel(
        _main_body,
        out_type=jax.ShapeDtypeStruct((N_PTS, N_OUT), jnp.float32),
        mesh=mesh,
        scratch_types=[
            pltpu.VMEM((3 * PPW,), jnp.float32),
            pltpu.VMEM((24 * CP,), jnp.int32),
            pltpu.VMEM((9 * CP,), jnp.float32),
            pltpu.VMEM((8, CP, N_FEAT), jnp.float32),
            pltpu.VMEM((8, CP, N_FEAT), jnp.float32),
            pltpu.VMEM((8, CP, N_FEAT), jnp.float32),
            pltpu.VMEM((CP, N_OUT), jnp.float32),
            pltpu.SemaphoreType.DMA,
        ],
        **_SC_PARAMS,
    )
    return relayout, main


def kernel(x, f0, f1, f2):
    relayout, main = _build()
    xflat = x.T.reshape(-1)
    ffs = [f.reshape(-1) for f in (f0, f1, f2)]
    tab = relayout(*ffs).reshape(NROWS, N_FEAT)
    return main(xflat, tab)


# speedup vs baseline: 36.4603x; 1.2075x over previous
import functools
import math

import jax
import jax.numpy as jnp
from jax import lax
from jax.experimental import pallas as pl
from jax.experimental.pallas import tpu as pltpu, tpu_sc as plsc

N_PTS = 262144
N_FEAT = 8
RES = (32, 64, 128)
NVOX = tuple(r * r * r for r in RES)
OFS = (0, NVOX[0], NVOX[0] + NVOX[1])
NROWS = NVOX[0] + NVOX[1] + NVOX[2]
L = 4
N_OUT = 2 * L * 3 + 3 * N_FEAT

NW = 32
PPW = N_PTS // NW
CP = 128
NCHUNK = PPW // CP
NGROUP = CP // 16

PIO2 = math.pi * 0.5
INV_PIO2 = 2.0 / math.pi
S1, S2, S3 = -0.16666667, 0.0083333310, -0.00019840874
C1, C2, C3 = -0.5, 0.041666638, -0.0013888378

_SC_PARAMS = dict(
    compiler_params=pltpu.CompilerParams(
        needs_layout_passes=False, use_tc_tiling_on_sc=False),
)


def _wid():
    return lax.axis_index("s") * 2 + lax.axis_index("c")


def _relayout_body(ff0, ff1, ff2, tout, chbuf, obuf, rsem):
    wid = _wid()
    iota = lax.iota(jnp.int32, 16)
    lane_c = iota & 7
    lane_p = lax.shift_right_logical(iota, 3)
    srcs = (ff0, ff1, ff2)
    for gi in range(3):
        nv = NVOX[gi]
        span = nv // NW
        k = min(span, 4096)
        nck = span // k
        const = lane_c * k + lane_p
        wbase = wid * span

        @pl.loop(0, nck)
        def _chunk(ci, _gi=gi, _k=k, _const=const, _wbase=wbase, _nv=nv):
            pos0 = _wbase + ci * _k
            descs = [pltpu.async_copy(
                srcs[_gi].at[pl.ds(ch * _nv + pos0, _k)],
                chbuf.at[pl.ds(ch * _k, _k)], rsem) for ch in range(8)]
            for d in descs:
                d.wait()

            @pl.loop(0, _k // 16, unroll=4)
            def _pair(j):
                for u in range(8):
                    v = plsc.load_gather(chbuf, [_const + (j * 16 + u * 2)])
                    obuf[pl.ds(j * 128 + u * 16, 16)] = v

            pltpu.sync_copy(
                obuf.at[pl.ds(0, _k * 8)],
                tout.at[pl.ds((OFS[_gi] + pos0) * 8, _k * 8)])


def _splat(v):
    return jnp.full((16,), v, jnp.int32)


def _main_body(xflat, tab, out,
               xbuf, idxbuf, wbuf, da0, da1, da2, db0, db1, db2, staging,
               sem_a, sem_b):
    base = _wid() * PPW
    for j in range(3):
        pltpu.sync_copy(xflat.at[pl.ds(j * N_PTS + base, PPW)],
                        xbuf.at[pl.ds(j * PPW, PPW)])

    iota = lax.iota(jnp.int32, 16)
    dset = ((da0, da1, da2), (db0, db1, db2))
    sems = (sem_a, sem_b)

    def fire(c, s):
        xoff = c * CP
        ib = s * 24 * CP
        wb = s * 9 * CP

        def idx_group(g, carry):
            off = xoff + g * 16
            xv = [xbuf[pl.ds(j * PPW + off, 16)] for j in range(3)]
            xp = [v + 1.0 for v in xv]
            for gi in range(3):
                r = RES[gi]
                sc = 0.5 * (r - 1)
                p = [v * sc for v in xp]
                f = [v.astype(jnp.int32) for v in p]
                w = [p[j] - f[j].astype(jnp.float32) for j in range(3)]
                i0 = [jnp.minimum(v, r - 1) for v in f]
                i1 = [jnp.minimum(v + 1, r - 1) for v in i0]
                for j in range(3):
                    wbuf[pl.ds(wb + (3 * gi + j) * CP + g * 16, 16)] = w[j]
                zb = (i0[2] * (r * r) + OFS[gi], i1[2] * (r * r) + OFS[gi])
                yb = (i0[1] * r, i1[1] * r)
                xb = (i0[0], i1[0])
                k = 0
                for dz in range(2):
                    for dy in range(2):
                        for dx in range(2):
                            idxbuf[pl.ds(ib + (gi * 8 + k) * CP + g * 16, 16)] = (
                                zb[dz] + yb[dy] + xb[dx])
                            k += 1
            return carry

        lax.fori_loop(0, NGROUP, idx_group, 0)
        for gi in range(3):
            for k in range(8):
                pltpu.async_copy(
                    tab.at[idxbuf.at[pl.ds(ib + (gi * 8 + k) * CP, CP)]],
                    dset[s][gi].at[k], sems[s])

    def wait_set(s):
        for gi in range(3):
            for k in range(8):
                pltpu.make_async_copy(
                    tab.at[pl.ds(0, CP)], dset[s][gi].at[k], sems[s]).wait()

    def blend(c, s):
        xoff = c * CP
        wb = s * 9 * CP
        dests = dset[s]

        def blend_group(g, carry):
            off = xoff + g * 16
            rowv = g * 16 + iota
            xv = [xbuf[pl.ds(j * PPW + off, 16)] for j in range(3)]
            for l in range(L):
                coef = (l + 1) * PIO2
                for j in range(3):
                    arg = xv[j] * coef
                    t = arg * INV_PIO2
                    k8 = (t + 8.5).astype(jnp.int32)
                    kf = k8.astype(jnp.float32) - 8.0
                    rr = arg - kf * PIO2
                    r2 = rr * rr
                    sp = rr + rr * (r2 * (S1 + r2 * (S2 + r2 * S3)))
                    cp = 1.0 + r2 * (C1 + r2 * (C2 + r2 * C3))
                    swap = (k8 & 1) == 1
                    sv = jnp.where(swap, cp, sp)
                    cv = jnp.where(swap, sp, cp)
                    so = jnp.where((k8 & 2) == 2, -sv, sv)
                    co = jnp.where(((k8 + 1) & 2) == 2, -cv, cv)
                    col = l * 3 + j
                    plsc.store_scatter(staging, [rowv, _splat(col)], so)
                    plsc.store_scatter(staging, [rowv, _splat(12 + col)], co)
            for gi in range(3):
                wx = wbuf[pl.ds(wb + (3 * gi + 0) * CP + g * 16, 16)]
                wy = wbuf[pl.ds(wb + (3 * gi + 1) * CP + g * 16, 16)]
                wz = wbuf[pl.ds(wb + (3 * gi + 2) * CP + g * 16, 16)]
                dg = dests[gi]
                for ch in range(8):
                    chv = _splat(ch)
                    c = [plsc.load_gather(dg, [_splat(k), rowv, chv])
                         for k in range(8)]
                    c00 = c[0] + wx * (c[1] - c[0])
                    c01 = c[2] + wx * (c[3] - c[2])
                    c10 = c[4] + wx * (c[5] - c[4])
                    c11 = c[6] + wx * (c[7] - c[6])
                    c0 = c00 + wy * (c01 - c00)
                    c1 = c10 + wy * (c11 - c10)
                    o = c0 + wz * (c1 - c0)
                    plsc.store_scatter(
                        staging, [rowv, _splat(24 + 8 * gi + ch)], o)
            return carry

        lax.fori_loop(0, NGROUP, blend_group, 0)

        pltpu.sync_copy(staging, out.at[pl.ds(base + xoff, CP)])

    fire(0, 0)

    @pl.loop(0, NCHUNK // 2)
    def piperound(ci):
        c0 = ci * 2
        fire(c0 + 1, 1)
        wait_set(0)
        blend(c0, 0)

        @pl.when(c0 + 2 < NCHUNK)
        def _():
            fire(c0 + 2, 0)

        wait_set(1)
        blend(c0 + 1, 1)


@functools.cache
def _build():
    mesh = plsc.VectorSubcoreMesh(core_axis_name="c", subcore_axis_name="s")
    relayout = pl.kernel(
        _relayout_body,
        out_type=jax.ShapeDtypeStruct((NROWS * N_FEAT,), jnp.float32),
        mesh=mesh,
        scratch_types=[
            pltpu.VMEM((8 * 4096,), jnp.float32),
            pltpu.VMEM((4096 * 8,), jnp.float32),
            pltpu.SemaphoreType.DMA,
        ],
        **_SC_PARAMS,
    )
    main = pl.kernel(
        _main_body,
        out_type=jax.ShapeDtypeStruct((N_PTS, N_OUT), jnp.float32),
        mesh=mesh,
        scratch_types=[
            pltpu.VMEM((3 * PPW,), jnp.float32),
            pltpu.VMEM((2 * 24 * CP,), jnp.int32),
            pltpu.VMEM((2 * 9 * CP,), jnp.float32),
            pltpu.VMEM((8, CP, N_FEAT), jnp.float32),
            pltpu.VMEM((8, CP, N_FEAT), jnp.float32),
            pltpu.VMEM((8, CP, N_FEAT), jnp.float32),
            pltpu.VMEM((8, CP, N_FEAT), jnp.float32),
            pltpu.VMEM((8, CP, N_FEAT), jnp.float32),
            pltpu.VMEM((8, CP, N_FEAT), jnp.float32),
            pltpu.VMEM((CP, N_OUT), jnp.float32),
            pltpu.SemaphoreType.DMA,
            pltpu.SemaphoreType.DMA,
        ],
        **_SC_PARAMS,
    )
    return relayout, main


def kernel(x, f0, f1, f2):
    relayout, main = _build()
    xflat = x.T.reshape(-1)
    ffs = [f.reshape(-1) for f in (f0, f1, f2)]
    tab = relayout(*ffs).reshape(NROWS, N_FEAT)
    return main(xflat, tab)
